# baseline (speedup 1.0000x reference)
import jax
import jax.numpy as jnp
from jax.experimental import pallas as pl

ALPHA = 1.702
LIMIT = 7.0
FC = 512


def _moe_body(xt_ref, wg_ref, wu_ref, w2t_ref, rw_ref, b1g_ref, b1u_ref,
              b2_ref, out_ref):
    e = pl.program_id(0)

    @pl.when(e == 0)
    def _init():
        out_ref[...] = jnp.zeros_like(out_ref)

    xt = xt_ref[...]
    f = wg_ref.shape[1]
    rw_row = rw_ref[0]
    for c in range(f // FC):
        sl = pl.ds(c * FC, FC)
        g = jnp.dot(wg_ref[0, sl, :], xt,
                    preferred_element_type=jnp.float32)
        u = jnp.dot(wu_ref[0, sl, :], xt,
                    preferred_element_type=jnp.float32)
        g = g + b1g_ref[0, sl, :]
        u = u + b1u_ref[0, sl, :]
        g = jnp.minimum(g, LIMIT)
        u = jnp.clip(u, -LIMIT, LIMIT)
        glu = g * jax.nn.sigmoid(g * ALPHA)
        ht = ((u + 1.0) * glu * rw_row).astype(jnp.bfloat16)
        out_ref[...] += jnp.dot(w2t_ref[0, :, sl], ht,
                                preferred_element_type=jnp.float32)
    out_ref[...] += b2_ref[0] * rw_row


@jax.jit
def kernel(hidden_states, router_indices, routing_weights, gate_up_proj,
           gate_up_proj_bias, down_proj, down_proj_bias):
    bsz, tt, hid = hidden_states.shape
    num_e, _, f2 = gate_up_proj.shape
    f = f2 // 2
    tok = bsz * tt

    xt = jnp.swapaxes(hidden_states.reshape(tok, hid), 0, 1).astype(
        jnp.bfloat16)
    w1t = jnp.swapaxes(gate_up_proj, 1, 2).astype(jnp.bfloat16)
    wg = w1t[:, 0::2, :]
    wu = w1t[:, 1::2, :]
    w2t = jnp.swapaxes(down_proj, 1, 2).astype(jnp.bfloat16)
    b1g = gate_up_proj_bias[:, 0::2].reshape(num_e, f, 1)
    b1u = gate_up_proj_bias[:, 1::2].reshape(num_e, f, 1)
    b2r = down_proj_bias.reshape(num_e, hid, 1)
    rw = routing_weights.T.reshape(num_e, 1, tok)

    out_t = pl.pallas_call(
        _moe_body,
        grid=(num_e,),
        in_specs=[
            pl.BlockSpec((hid, tok), lambda e: (0, 0)),
            pl.BlockSpec((1, f, hid), lambda e: (e, 0, 0)),
            pl.BlockSpec((1, f, hid), lambda e: (e, 0, 0)),
            pl.BlockSpec((1, hid, f), lambda e: (e, 0, 0)),
            pl.BlockSpec((1, 1, tok), lambda e: (e, 0, 0)),
            pl.BlockSpec((1, f, 1), lambda e: (e, 0, 0)),
            pl.BlockSpec((1, f, 1), lambda e: (e, 0, 0)),
            pl.BlockSpec((1, hid, 1), lambda e: (e, 0, 0)),
        ],
        out_specs=pl.BlockSpec((hid, tok), lambda e: (0, 0)),
        out_shape=jax.ShapeDtypeStruct((hid, tok), jnp.float32),
    )(xt, wg, wu, w2t, rw, b1g, b1u, b2r)
    return jnp.swapaxes(out_t, 0, 1).reshape(bsz, tt, hid)
